# baseline (speedup 1.0000x reference)
import functools

import jax
import jax.numpy as jnp
from jax import lax
from jax.experimental import pallas as pl
from jax.experimental.pallas import tpu as pltpu
from jax.experimental.pallas import tpu_sc as plsc

HIDDEN = 64
CHUNK = 128
NBUF = 8


@functools.lru_cache(maxsize=None)
def _build(Bt, L, V):
    info = plsc.get_sparse_core_info()
    NC, NS = info.num_cores, info.num_subcores
    NW = NC * NS
    rpw = Bt // NW
    bpw = rpw * L
    nch = bpw // CHUNK
    iters = nch // NBUF
    mesh = plsc.VectorSubcoreMesh(core_axis_name="c", subcore_axis_name="s")

    @functools.partial(
        pl.kernel,
        mesh=mesh,
        compiler_params=pltpu.CompilerParams(use_tc_tiling_on_sc=False),
        out_type=jax.ShapeDtypeStruct((Bt * L, HIDDEN), jnp.float32),
        scratch_types=(
            [pltpu.VMEM((rpw, L), jnp.int32), pltpu.VMEM((bpw,), jnp.int32)]
            + [pltpu.VMEM((CHUNK, HIDDEN), jnp.float32) for _ in range(NBUF)]
            + [pltpu.SemaphoreType.DMA for _ in range(NBUF)]
        ),
    )
    def k(lab_hbm, table_hbm, out_hbm, lab_v, idx_v, *rest):
        bufs = rest[:NBUF]
        sems = rest[NBUF:]
        wid = lax.axis_index("s") * NC + lax.axis_index("c")
        base = wid * bpw
        pltpu.sync_copy(lab_hbm.at[pl.ds(wid * rpw, rpw)], lab_v)

        def repack(r, carry):
            idx_v[pl.ds(r * L, 16)] = lab_v[r, pl.ds(0, 16)]
            idx_v[pl.ds(r * L + L - 16, 16)] = lab_v[r, pl.ds(L - 16, 16)]
            return carry

        lax.fori_loop(0, rpw, repack, None)

        def body(o, carry):
            c0 = o * NBUF
            g = [
                pltpu.async_copy(
                    table_hbm.at[idx_v.at[pl.ds((c0 + i) * CHUNK, CHUNK)]],
                    bufs[i],
                    sems[i],
                )
                for i in range(NBUF)
            ]
            st = []
            for i in range(NBUF):
                g[i].wait()
                st.append(
                    pltpu.async_copy(
                        bufs[i],
                        out_hbm.at[pl.ds(base + (c0 + i) * CHUNK, CHUNK)],
                        sems[i],
                    )
                )
            for cp in st:
                cp.wait()
            return carry

        lax.fori_loop(0, iters, body, None)

    return k


def kernel(labels, train, table):
    Bt, L = labels.shape
    k = _build(Bt, L, table.shape[0])
    out = k(labels, table)
    return out.reshape(Bt, L * HIDDEN)


# speedup vs baseline: 1.0017x; 1.0017x over previous
import functools

import jax
import jax.numpy as jnp
from jax import lax
from jax.experimental import pallas as pl
from jax.experimental.pallas import tpu as pltpu
from jax.experimental.pallas import tpu_sc as plsc

HIDDEN = 64
LPAD = 128
CHUNK = 128
NBUF = 8


@functools.lru_cache(maxsize=None)
def _build(Bt, L, V):
    info = plsc.get_sparse_core_info()
    NC, NS = info.num_cores, info.num_subcores
    NW = NC * NS
    rpw = Bt // NW
    bpw = rpw * L
    nch = bpw // CHUNK
    iters = nch // NBUF
    mesh = plsc.VectorSubcoreMesh(core_axis_name="c", subcore_axis_name="s")

    @functools.partial(
        pl.kernel,
        mesh=mesh,
        compiler_params=pltpu.CompilerParams(use_tc_tiling_on_sc=False),
        out_type=jax.ShapeDtypeStruct((Bt * L, HIDDEN), jnp.float32),
        scratch_types=(
            [pltpu.VMEM((rpw, 32), jnp.int32), pltpu.VMEM((bpw,), jnp.int32)]
            + [pltpu.VMEM((CHUNK, HIDDEN), jnp.float32) for _ in range(NBUF)]
            + [pltpu.SemaphoreType.DMA for _ in range(NBUF)]
        ),
    )
    def k(lab_hbm, table_hbm, out_hbm, lab_v, idx_v, *rest):
        bufs = rest[:NBUF]
        sems = rest[NBUF:]
        wid = lax.axis_index("s") * NC + lax.axis_index("c")
        base = wid * bpw
        pltpu.sync_copy(lab_hbm.at[pl.ds(wid * rpw, rpw), pl.ds(0, 32)], lab_v)

        def repack(r, carry):
            idx_v[pl.ds(r * L, 16)] = lab_v[r, pl.ds(0, 16)]
            idx_v[pl.ds(r * L + L - 16, 16)] = lab_v[r, pl.ds(L - 16, 16)]
            return carry

        lax.fori_loop(0, rpw, repack, None)

        def body(o, carry):
            c0 = o * NBUF
            g = [
                pltpu.async_copy(
                    table_hbm.at[idx_v.at[pl.ds((c0 + i) * CHUNK, CHUNK)]],
                    bufs[i],
                    sems[i],
                )
                for i in range(NBUF)
            ]
            st = []
            for i in range(NBUF):
                g[i].wait()
                st.append(
                    pltpu.async_copy(
                        bufs[i],
                        out_hbm.at[pl.ds(base + (c0 + i) * CHUNK, CHUNK)],
                        sems[i],
                    )
                )
            for cp in st:
                cp.wait()
            return carry

        lax.fori_loop(0, iters, body, None)

    return k


def kernel(labels, train, table):
    Bt, L = labels.shape
    lab_pad = jnp.pad(labels, ((0, 0), (0, LPAD - L)))
    k = _build(Bt, L, table.shape[0])
    out = k(lab_pad, table)
    return out.reshape(Bt, L * HIDDEN)


# speedup vs baseline: 1.0050x; 1.0033x over previous
import functools

import jax
import jax.numpy as jnp
from jax import lax
from jax.experimental import pallas as pl
from jax.experimental.pallas import tpu as pltpu
from jax.experimental.pallas import tpu_sc as plsc

HIDDEN = 64
LPAD = 128
CHUNK = 128
NBUF = 8


@functools.lru_cache(maxsize=None)
def _build(Bt, L, V):
    info = plsc.get_sparse_core_info()
    NC, NS = info.num_cores, info.num_subcores
    NW = NC * NS
    rpw = Bt // NW
    bpw = rpw * L
    nch = bpw // CHUNK
    iters = nch // NBUF
    mesh = plsc.VectorSubcoreMesh(core_axis_name="c", subcore_axis_name="s")

    @functools.partial(
        pl.kernel,
        mesh=mesh,
        compiler_params=pltpu.CompilerParams(use_tc_tiling_on_sc=False),
        out_type=jax.ShapeDtypeStruct((Bt * L, HIDDEN), jnp.float32),
        scratch_types=(
            [pltpu.VMEM((rpw, 32), jnp.int32), pltpu.VMEM((bpw,), jnp.int32)]
            + [pltpu.VMEM((CHUNK, HIDDEN), jnp.float32) for _ in range(NBUF)]
            + [pltpu.SemaphoreType.DMA for _ in range(NBUF)]
        ),
    )
    def k(lab_hbm, table_hbm, out_hbm, lab_v, idx_v, *rest):
        bufs = rest[:NBUF]
        sems = rest[NBUF:]
        wid = lax.axis_index("s") * NC + lax.axis_index("c")
        base = wid * bpw
        pltpu.sync_copy(lab_hbm.at[pl.ds(wid * rpw, rpw), pl.ds(0, 32)], lab_v)

        def repack(r, carry):
            idx_v[pl.ds(r * L, 16)] = lab_v[r, pl.ds(0, 16)]
            idx_v[pl.ds(r * L + L - 16, 16)] = lab_v[r, pl.ds(L - 16, 16)]
            return carry

        lax.fori_loop(0, rpw, repack, None)

        def body(o, carry):
            c0 = o * NBUF
            g = [
                pltpu.async_copy(
                    table_hbm.at[idx_v.at[pl.ds((c0 + i) * CHUNK, CHUNK)]],
                    bufs[i],
                    sems[i],
                )
                for i in range(NBUF)
            ]
            st = []
            for i in range(NBUF):
                g[i].wait()
                st.append(
                    pltpu.async_copy(
                        bufs[i],
                        out_hbm.at[pl.ds(base + (c0 + i) * CHUNK, CHUNK)],
                        sems[i],
                    )
                )
            for cp in st:
                cp.wait()
            return carry

        lax.fori_loop(0, iters, body, None)

    return k


def kernel(labels, train, table):
    Bt, L = labels.shape
    lab_pad = jnp.pad(labels, ((0, 0), (0, LPAD - L)))
    V = table.shape[0] - 1
    t2 = table[:V].reshape(V // 2, 2 * HIDDEN)
    t2 = jax.lax.optimization_barrier(t2)
    t3 = t2.reshape(V, HIDDEN)
    k = _build(Bt, L, V)
    out = k(lab_pad, t3)
    return out.reshape(Bt, L * HIDDEN)


# speedup vs baseline: 1.1265x; 1.1209x over previous
import functools

import jax
import jax.numpy as jnp
from jax import lax
from jax.experimental import pallas as pl
from jax.experimental.pallas import tpu as pltpu
from jax.experimental.pallas import tpu_sc as plsc

HIDDEN = 64
LPAD = 128
CHUNK = 128
NBUF = 8


@functools.lru_cache(maxsize=None)
def _build(Bt, L, V):
    info = plsc.get_sparse_core_info()
    NC, NS = info.num_cores, info.num_subcores
    NW = NC * NS
    rpw = Bt // NW
    bpw = rpw * L
    nch = bpw // CHUNK
    iters = nch // NBUF
    ntile = L * HIDDEN // 128
    mesh = plsc.VectorSubcoreMesh(core_axis_name="c", subcore_axis_name="s")

    @functools.partial(
        pl.kernel,
        mesh=mesh,
        compiler_params=pltpu.CompilerParams(use_tc_tiling_on_sc=False),
        out_type=jax.ShapeDtypeStruct((Bt * L, HIDDEN), jnp.float32),
        scratch_types=(
            [
                pltpu.VMEM((rpw, 32), jnp.int32),
                pltpu.VMEM((bpw,), jnp.int32),
                pltpu.VMEM((nch, CHUNK), jnp.int32),
            ]
            + [pltpu.VMEM((CHUNK, HIDDEN), jnp.float32) for _ in range(NBUF)]
            + [pltpu.SemaphoreType.DMA for _ in range(NBUF)]
        ),
    )
    def k(lab_hbm, table_hbm, out_hbm, lab_v, idx_v, q_v, *rest):
        bufs = rest[:NBUF]
        sems = rest[NBUF:]
        wid = lax.axis_index("s") * NC + lax.axis_index("c")
        pltpu.sync_copy(lab_hbm.at[pl.ds(wid * rpw, rpw), pl.ds(0, 32)], lab_v)

        def repack(r, carry):
            idx_v[pl.ds(r * L, 16)] = lab_v[r, pl.ds(0, 16)]
            idx_v[pl.ds(r * L + L - 16, 16)] = lab_v[r, pl.ds(L - 16, 16)]
            return carry

        lax.fori_loop(0, rpw, repack, None)

        lanes = lax.iota(jnp.int32, 16)

        def qrow(c, carry):
            for kk in range(CHUNK // 16):
                b = c * CHUNK + kk * 16 + lanes
                r = jax.lax.shift_right_logical(b * 3277, 16)
                j = b - r * L
                q = (
                    wid * bpw
                    + jax.lax.shift_right_logical(r, 3) * (16 * ntile)
                    + (r & 7) * 2
                    + jax.lax.shift_right_logical(j, 1) * 16
                    + (j & 1)
                )
                q_v[c, pl.ds(kk * 16, 16)] = q
            return carry

        lax.fori_loop(0, nch, qrow, None)

        def body(o, carry):
            c0 = o * NBUF
            g = [
                pltpu.async_copy(
                    table_hbm.at[idx_v.at[pl.ds((c0 + i) * CHUNK, CHUNK)]],
                    bufs[i],
                    sems[i],
                )
                for i in range(NBUF)
            ]
            st = []
            for i in range(NBUF):
                g[i].wait()
                st.append(
                    pltpu.async_copy(
                        bufs[i],
                        out_hbm.at[q_v.at[c0 + i]],
                        sems[i],
                    )
                )
            for cp in st:
                cp.wait()
            return carry

        lax.fori_loop(0, iters, body, None)

    return k


def kernel(labels, train, table):
    Bt, L = labels.shape
    lab_pad = jnp.pad(labels, ((0, 0), (0, LPAD - L)))
    V = table.shape[0] - 1
    t2 = table[:V].reshape(V // 2, 2 * HIDDEN)
    t2 = jax.lax.optimization_barrier(t2)
    t3 = t2.reshape(V, HIDDEN)
    k = _build(Bt, L, V)
    out = k(lab_pad, t3)
    ntile = L * HIDDEN // 128
    out4 = out.reshape(Bt // 8, ntile, 8, 128)
    return out4.transpose(0, 2, 1, 3).reshape(Bt, L * HIDDEN)
